# baseline (speedup 1.0000x reference)
import jax
import jax.numpy as jnp
from jax.experimental import pallas as pl
from jax.experimental.pallas import tpu as pltpu

B, S, D, F, T = 2, 4096, 1024, 4096, 4
M = B * S

TFP = 512
NFP = F // TFP

TM = 1024
NM = M // TM


def _prep_kernel(tid_ref, win_ref, dwi_ref, wout_ref, dwo_ref,
                 bin_ref, dbi_ref, bout_ref, dbo_ref,
                 w1_out, w2_out, b1_out, b2_out):
    w1_out[...] = (win_ref[...] + dwi_ref[0]).astype(jnp.bfloat16)
    w2_out[...] = (wout_ref[...] + dwo_ref[0]).astype(jnp.bfloat16)
    b1_out[...] = bin_ref[...] + dbi_ref[0]
    b2_out[...] = bout_ref[...] + dbo_ref[0]


TC = 1024
NC = F // TC


def _ffn_kernel(xb_ref, b1_ref, b2_ref, *rest):
    w1_refs = rest[:NC]
    w2_refs = rest[NC:2 * NC]
    out_ref = rest[2 * NC]
    xb = xb_ref[...].astype(jnp.bfloat16)
    acc = b2_ref[...]
    for c in range(NC):
        h = jax.lax.dot_general(
            xb, w1_refs[c][...], (((1,), (1,)), ((), ())),
            preferred_element_type=jnp.float32)
        h = jax.nn.gelu(h + b1_ref[:, c * TC:(c + 1) * TC])
        acc = acc + jax.lax.dot_general(
            h.astype(jnp.bfloat16), w2_refs[c][...], (((1,), (1,)), ((), ())),
            preferred_element_type=jnp.float32)
    out_ref[...] = acc


def kernel(x, W_in, b_in, W_out, b_out, dW_in, db_in, dW_out, db_out, task_id):
    xm = x.reshape(M, D)
    b_in2 = b_in.reshape(1, F)
    b_out2 = b_out.reshape(1, D)
    db_in3 = db_in.reshape(T, 1, F)
    db_out3 = db_out.reshape(T, 1, D)
    tid = jnp.asarray(task_id, jnp.int32).reshape(1)

    prep_spec = pltpu.PrefetchScalarGridSpec(
        num_scalar_prefetch=1,
        grid=(NFP,),
        in_specs=[
            pl.BlockSpec((TFP, D), lambda f, t: (f, 0)),
            pl.BlockSpec((1, TFP, D), lambda f, t: (t[0], f, 0)),
            pl.BlockSpec((D, TFP), lambda f, t: (0, f)),
            pl.BlockSpec((1, D, TFP), lambda f, t: (t[0], 0, f)),
            pl.BlockSpec((1, TFP), lambda f, t: (0, f)),
            pl.BlockSpec((1, 1, TFP), lambda f, t: (t[0], 0, f)),
            pl.BlockSpec((1, D), lambda f, t: (0, 0)),
            pl.BlockSpec((1, 1, D), lambda f, t: (t[0], 0, 0)),
        ],
        out_specs=[
            pl.BlockSpec((TFP, D), lambda f, t: (f, 0)),
            pl.BlockSpec((D, TFP), lambda f, t: (0, f)),
            pl.BlockSpec((1, TFP), lambda f, t: (0, f)),
            pl.BlockSpec((1, D), lambda f, t: (0, 0)),
        ],
    )
    w1e, w2e, b1e, b2e = pl.pallas_call(
        _prep_kernel,
        grid_spec=prep_spec,
        out_shape=[
            jax.ShapeDtypeStruct((F, D), jnp.bfloat16),
            jax.ShapeDtypeStruct((D, F), jnp.bfloat16),
            jax.ShapeDtypeStruct((1, F), jnp.float32),
            jax.ShapeDtypeStruct((1, D), jnp.float32),
        ],
    )(tid, W_in, dW_in, W_out, dW_out, b_in2, db_in3, b_out2, db_out3)

    w1_specs = [pl.BlockSpec((TC, D), lambda m, c=c: (c, 0)) for c in range(NC)]
    w2_specs = [pl.BlockSpec((D, TC), lambda m, c=c: (0, c)) for c in range(NC)]
    out = pl.pallas_call(
        _ffn_kernel,
        grid=(NM,),
        in_specs=[
            pl.BlockSpec((TM, D), lambda m: (m, 0)),
            pl.BlockSpec((1, F), lambda m: (0, 0)),
            pl.BlockSpec((1, D), lambda m: (0, 0)),
        ] + w1_specs + w2_specs,
        out_specs=pl.BlockSpec((TM, D), lambda m: (m, 0)),
        out_shape=jax.ShapeDtypeStruct((M, D), jnp.float32),
        compiler_params=pltpu.CompilerParams(
            dimension_semantics=("arbitrary",)),
    )(xm, b1e, b2e, *([w1e] * NC), *([w2e] * NC))
    return out.reshape(B, S, D)


# speedup vs baseline: 1.0518x; 1.0518x over previous
import jax
import jax.numpy as jnp
from jax.experimental import pallas as pl
from jax.experimental.pallas import tpu as pltpu

B, S, D, F, T = 2, 4096, 1024, 4096, 4
M = B * S

TFP = 256
NFP = F // TFP
TM = 1024
NM = M // TM
TC = 1024
NC = F // TC


def _kernel(tid_ref, win_ref, dwi_ref, wout_ref, dwo_ref,
            bin_ref, dbi_ref, bout_ref, dbo_ref, x_ref, out_ref,
            w1_scr, w2_scr, b1_scr, b2_scr):
    i = pl.program_id(0)

    @pl.when(i < NFP)
    def _prep():
        fo = i * TFP
        w1_scr[pl.ds(fo, TFP), :] = (
            win_ref[...] + dwi_ref[0]).astype(jnp.bfloat16)
        w2_scr[:, pl.ds(fo, TFP)] = (
            wout_ref[...] + dwo_ref[0]).astype(jnp.bfloat16)
        b1_scr[:, pl.ds(fo, TFP)] = bin_ref[...] + dbi_ref[0]

        @pl.when(i == 0)
        def _():
            b2_scr[...] = bout_ref[...] + dbo_ref[0]

    @pl.when(i >= NFP)
    def _compute():
        xb = x_ref[...].astype(jnp.bfloat16)
        acc = b2_scr[...]
        for c in range(NC):
            w1 = w1_scr[pl.ds(c * TC, TC), :]
            h = jax.lax.dot_general(
                xb, w1, (((1,), (1,)), ((), ())),
                preferred_element_type=jnp.float32)
            h = jax.nn.gelu(h + b1_scr[:, c * TC:(c + 1) * TC])
            w2 = w2_scr[:, pl.ds(c * TC, TC)]
            acc = acc + jax.lax.dot_general(
                h.astype(jnp.bfloat16), w2, (((1,), (1,)), ((), ())),
                preferred_element_type=jnp.float32)
        out_ref[...] = acc


def kernel(x, W_in, b_in, W_out, b_out, dW_in, db_in, dW_out, db_out, task_id):
    xm = x.reshape(M, D)
    b_in2 = b_in.reshape(1, F)
    b_out2 = b_out.reshape(1, D)
    db_in3 = db_in.reshape(T, 1, F)
    db_out3 = db_out.reshape(T, 1, D)
    tid = jnp.asarray(task_id, jnp.int32).reshape(1)

    def fw(i):
        return jnp.minimum(i, NFP - 1)

    def mw(i):
        return jnp.maximum(i - NFP, 0)

    grid_spec = pltpu.PrefetchScalarGridSpec(
        num_scalar_prefetch=1,
        grid=(NFP + NM,),
        in_specs=[
            pl.BlockSpec((TFP, D), lambda i, t: (fw(i), 0)),
            pl.BlockSpec((1, TFP, D), lambda i, t: (t[0], fw(i), 0)),
            pl.BlockSpec((D, TFP), lambda i, t: (0, fw(i))),
            pl.BlockSpec((1, D, TFP), lambda i, t: (t[0], 0, fw(i))),
            pl.BlockSpec((1, TFP), lambda i, t: (0, fw(i))),
            pl.BlockSpec((1, 1, TFP), lambda i, t: (t[0], 0, fw(i))),
            pl.BlockSpec((1, D), lambda i, t: (0, 0)),
            pl.BlockSpec((1, 1, D), lambda i, t: (t[0], 0, 0)),
            pl.BlockSpec((TM, D), lambda i, t: (mw(i), 0)),
        ],
        out_specs=pl.BlockSpec((TM, D), lambda i, t: (mw(i), 0)),
        scratch_shapes=[
            pltpu.VMEM((F, D), jnp.bfloat16),
            pltpu.VMEM((D, F), jnp.bfloat16),
            pltpu.VMEM((1, F), jnp.float32),
            pltpu.VMEM((1, D), jnp.float32),
        ],
    )
    out = pl.pallas_call(
        _kernel,
        grid_spec=grid_spec,
        out_shape=jax.ShapeDtypeStruct((M, D), jnp.float32),
        compiler_params=pltpu.CompilerParams(
            dimension_semantics=("arbitrary",)),
    )(tid, W_in, dW_in, W_out, dW_out, b_in2, db_in3, b_out2, db_out3, xm)
    return out.reshape(B, S, D)
